# baseline (speedup 1.0000x reference)
import functools
import math

import jax
import jax.numpy as jnp
from jax import lax
from jax.experimental import pallas as pl
from jax.experimental.pallas import tpu as pltpu
from jax.experimental.pallas import tpu_sc as plsc

NC = 2
NS = 16
NW = NC * NS
CH = 128

_F32 = jnp.float32


def _build_deg_kernel(n_nodes, d, n_chunks):
  rows_per_sub = n_nodes // NS
  zr = 32
  assert rows_per_sub % zr == 0 and rows_per_sub % 8 == 0
  chunks_per_sub = -(-n_chunks // NS)
  mesh = plsc.VectorSubcoreMesh(core_axis_name="c", subcore_axis_name="s")

  @functools.partial(
      pl.kernel,
      out_type=jax.ShapeDtypeStruct((NC, n_nodes, d), _F32),
      mesh=mesh,
      scratch_types=(
          [pltpu.VMEM((CH,), jnp.int32) for _ in range(3)] +
          [pltpu.VMEM((CH, d), _F32),
           pltpu.VMEM((zr, d), _F32),
           pltpu.VMEM_SHARED((n_nodes, d), _F32)] +
          [pltpu.SemaphoreType.DMA for _ in range(5)]
      ),
  )
  def deg_kernel(src_hbm, dst_hbm, deg_hbm, *refs):
    idx_v = refs[0:3]
    ones_v = refs[3]
    zb = refs[4]
    deg_sh = refs[5]
    sem_i = refs[6:9]
    sem_s = refs[9:11]

    cid = lax.axis_index("c")
    sid = lax.axis_index("s")
    base = sid * rows_per_sub

    def chunk_of(t):
      return sid + t * NS

    def guard(t, fn, *a):
      @pl.when(chunk_of(t) < n_chunks)
      def _():
        fn(*a)

    def issue_idx(t, bi):
      ch = chunk_of(t)

      @pl.when(cid == 0)
      def _():
        pltpu.async_copy(src_hbm.at[pl.ds(ch * CH, CH)], idx_v[bi], sem_i[bi])

      @pl.when(cid == 1)
      def _():
        pltpu.async_copy(dst_hbm.at[pl.ds(ch * CH, CH)], idx_v[bi], sem_i[bi])

    def issue_scatter(bi, bs):
      pltpu.make_async_copy(src_hbm.at[pl.ds(0, CH)], idx_v[bi], sem_i[bi]).wait()
      pltpu.async_copy(ones_v, deg_sh.at[idx_v[bi]], sem_s[bs], add=True)

    def drain_scatter(bi, bs):
      pltpu.make_async_copy(ones_v, deg_sh.at[idx_v[bi]], sem_s[bs]).wait()

    guard(0, issue_idx, 0, 0)
    guard(1, issue_idx, 1, 1)

    def fill_ones(i, c):
      for j in range(d // 16):
        ones_v[i, pl.ds(j * 16, 16)] = jnp.ones((16,), _F32)
      return c
    lax.fori_loop(0, CH, fill_ones, 0)

    def fill_zb(i, c):
      for j in range(d // 16):
        zb[i, pl.ds(j * 16, 16)] = jnp.zeros((16,), _F32)
      return c
    lax.fori_loop(0, zr, fill_zb, 0)

    def zero_slice(t, c):
      pltpu.sync_copy(zb, deg_sh.at[pl.ds(base + t * zr, zr)])
      return c
    lax.fori_loop(0, rows_per_sub // zr, zero_slice, 0)
    plsc.subcore_barrier()

    def step(t, bi, bs, drain=True):
      guard(t, issue_scatter, bi, bs)
      if drain:
        guard(t - 1, drain_scatter, (bi - 1) % 3, (bs - 1) % 2)
      guard(t + 2, issue_idx, t + 2, (bi + 2) % 3)

    step(0, 0, 0, drain=False)
    step(1, 1, 1)

    def edge_loop(i, c):
      t0 = 2 + i * 6
      for k in range(6):
        t = t0 + k

        @pl.when(chunk_of(t - 1) < n_chunks)
        def _(t=t, k=k):
          step(t, (2 + k) % 3, (2 + k) % 2)
      return c
    nsteps = chunks_per_sub - 1
    lax.fori_loop(0, (nsteps + 5) // 6, edge_loop, 0)
    plsc.subcore_barrier()

    pltpu.sync_copy(deg_sh.at[pl.ds(base, rows_per_sub)],
                    deg_hbm.at[cid, pl.ds(base, rows_per_sub)])

  return deg_kernel


def _build_agg_kernel(n_nodes, d, n_chunks, n_real):
  rows_per_sub = n_nodes // NS
  zr = 32
  assert rows_per_sub % zr == 0 and rows_per_sub % 8 == 0
  chunks_per_worker = -(-n_chunks // NW)
  mesh = plsc.VectorSubcoreMesh(core_axis_name="c", subcore_axis_name="s")

  @functools.partial(
      pl.kernel,
      out_type=jax.ShapeDtypeStruct((NC, n_nodes, d), _F32),
      mesh=mesh,
      scratch_types=[
          pltpu.VMEM((CH,), jnp.int32),
          pltpu.VMEM((CH,), jnp.int32),
          pltpu.VMEM((CH,), jnp.int32),
          pltpu.VMEM((CH,), jnp.int32),
          pltpu.VMEM((CH, d), _F32),
          pltpu.VMEM((CH, d), _F32),
          pltpu.VMEM((zr, d), _F32),
          pltpu.VMEM_SHARED((n_nodes, d), _F32),
          pltpu.SemaphoreType.DMA,
          pltpu.SemaphoreType.DMA,
          pltpu.SemaphoreType.DMA,
          pltpu.SemaphoreType.DMA,
      ],
  )
  def agg_kernel(x_hbm, src_hbm, dst_hbm, agg_hbm,
                 idx_s, idx_s2, idx_d, idx_d2, rows_v, rows_v2, zb, acc_sh,
                 sem_a, sem_b, sem_ia, sem_ib):
    cid = lax.axis_index("c")
    sid = lax.axis_index("s")
    wid = sid * NC + cid
    base = sid * rows_per_sub

    def issue_idx(ch, idxb, dstb, sem):
      pltpu.async_copy(src_hbm.at[pl.ds(ch * CH, CH)], idxb, sem)
      pltpu.async_copy(dst_hbm.at[pl.ds(ch * CH, CH)], dstb, sem)

    def wait_idx(idxb, dstb, sem):
      pltpu.make_async_copy(src_hbm.at[pl.ds(0, CH)], idxb, sem).wait()
      pltpu.make_async_copy(src_hbm.at[pl.ds(0, CH)], dstb, sem).wait()

    @pl.when(wid < n_chunks)
    def _():
      issue_idx(wid, idx_s, idx_d, sem_ia)

    @pl.when(wid + NW < n_chunks)
    def _():
      issue_idx(wid + NW, idx_s2, idx_d2, sem_ib)

    def fill_zb(i, c):
      for j in range(d // 16):
        zb[i, pl.ds(j * 16, 16)] = jnp.zeros((16,), _F32)
      return c
    lax.fori_loop(0, zr, fill_zb, 0)

    tail_sub = n_real // rows_per_sub
    tail_rows = n_real - tail_sub * rows_per_sub

    def zero_slice(t, c):
      pltpu.sync_copy(zb, acc_sh.at[pl.ds(base + t * zr, zr)])
      return c

    @pl.when(cid == 0)
    def _():
      @pl.when(sid < tail_sub)
      def _():
        pltpu.sync_copy(x_hbm.at[pl.ds(base, rows_per_sub)],
                        acc_sh.at[pl.ds(base, rows_per_sub)])

      @pl.when(sid == tail_sub)
      def _():
        if tail_rows:
          pltpu.sync_copy(x_hbm.at[pl.ds(tail_sub * rows_per_sub, tail_rows)],
                          acc_sh.at[pl.ds(base, tail_rows)])
        for t in range((rows_per_sub - tail_rows) // zr):
          pltpu.sync_copy(zb, acc_sh.at[pl.ds(base + tail_rows + t * zr, zr)])

      @pl.when(sid > tail_sub)
      def _():
        lax.fori_loop(0, rows_per_sub // zr, zero_slice, 0)

    @pl.when(cid == 1)
    def _():
      lax.fori_loop(0, rows_per_sub // zr, zero_slice, 0)

    @pl.when(wid < n_chunks)
    def _():
      wait_idx(idx_s, idx_d, sem_ia)
      pltpu.async_copy(x_hbm.at[idx_s], rows_v, sem_a)
    plsc.subcore_barrier()

    def step(c0, c1, c2, idxA, dstA, rowsA, semA, semIA,
             idxB, dstB, rowsB, semB, semIB):
      @pl.when(c1 < n_chunks)
      def _():
        wait_idx(idxB, dstB, semIB)
        pltpu.async_copy(x_hbm.at[idxB], rowsB, semB)

      @pl.when(c0 < n_chunks)
      def _():
        pltpu.make_async_copy(x_hbm.at[idxA], rowsA, semA).wait()
        pltpu.sync_copy(rowsA, acc_sh.at[dstA], add=True)

      @pl.when(c2 < n_chunks)
      def _():
        issue_idx(c2, idxA, dstA, semIA)

    def edge_loop(j, c):
      c0 = wid + (2 * j) * NW
      step(c0, c0 + NW, c0 + 2 * NW,
           idx_s, idx_d, rows_v, sem_a, sem_ia,
           idx_s2, idx_d2, rows_v2, sem_b, sem_ib)
      step(c0 + NW, c0 + 2 * NW, c0 + 3 * NW,
           idx_s2, idx_d2, rows_v2, sem_b, sem_ib,
           idx_s, idx_d, rows_v, sem_a, sem_ia)
      return c
    lax.fori_loop(0, (chunks_per_worker + 1) // 2, edge_loop, 0)
    plsc.subcore_barrier()

    pltpu.sync_copy(acc_sh.at[pl.ds(base, rows_per_sub)],
                    agg_hbm.at[cid, pl.ds(base, rows_per_sub)])

  return agg_kernel


def _scale_body(feat_ref, dego_ref, out_ref):
  deg = dego_ref[0, :, 0:1] + 1.0
  out_ref[...] = feat_ref[...] * lax.rsqrt(deg)


def _gelu_exact(x):
  return 0.5 * x * (1.0 + lax.erf(x * (1.0 / math.sqrt(2.0))))


def _mid_body(agg1_ref, deg_ref, w1_ref, b1_ref, w2_ref, out_ref):
  a1 = agg1_ref[0] + agg1_ref[1]
  ii = lax.rsqrt(deg_ref[1, :, 0:1] + 1.0)
  oo = lax.rsqrt(deg_ref[0, :, 0:1] + 1.0)
  h = jnp.dot(a1, w1_ref[...], preferred_element_type=_F32) * ii + b1_ref[...]
  h = _gelu_exact(h)
  out_ref[...] = jnp.dot(h * oo, w2_ref[...], preferred_element_type=_F32)


def _final_body(agg2_ref, degi_ref, b2_ref, out_ref):
  ii = lax.rsqrt(degi_ref[0, :, 0:1] + 1.0)
  y = (agg2_ref[0] + agg2_ref[1]) * ii + b2_ref[...]
  nrm = jnp.sqrt(jnp.sum(y * y, axis=1, keepdims=True))
  out_ref[...] = y / jnp.maximum(nrm, 1e-12)


def kernel(features, edge_index, W1, b1, W2, b2):
  n, d_in = features.shape
  d_h = W1.shape[1]
  d_out = W2.shape[1]
  e = edge_index.shape[1]
  assert e % CH == 0 and n % NS == 0
  n_chunks = e // CH

  src1d = edge_index[0]
  dst1d = edge_index[1]
  np_pad = -(-n // (NS * 64)) * (NS * 64)

  deg_k = _build_deg_kernel(np_pad, d_in, n_chunks)
  agg_k = _build_agg_kernel(np_pad, d_in, n_chunks, n)

  deg_p = deg_k(src1d, dst1d)

  br = 400
  grid = (n // br,)
  dego_spec = pl.BlockSpec((1, br, d_in), lambda i: (0, i, 0))
  degi_spec = pl.BlockSpec((1, br, d_in), lambda i: (1, i, 0))
  degb_spec = pl.BlockSpec((2, br, d_in), lambda i: (0, i, 0))
  row_spec = pl.BlockSpec((br, d_in), lambda i: (i, 0))
  agg_spec = pl.BlockSpec((2, br, d_in), lambda i: (0, i, 0))

  xs = pl.pallas_call(
      _scale_body,
      grid=grid,
      in_specs=[row_spec, dego_spec],
      out_specs=row_spec,
      out_shape=jax.ShapeDtypeStruct((n, d_in), _F32),
  )(features, deg_p)

  agg1_p = agg_k(xs, src1d, dst1d)

  m2 = pl.pallas_call(
      _mid_body,
      grid=grid,
      in_specs=[
          agg_spec,
          degb_spec,
          pl.BlockSpec((d_in, d_h), lambda i: (0, 0)),
          pl.BlockSpec((1, d_h), lambda i: (0, 0)),
          pl.BlockSpec((d_h, d_out), lambda i: (0, 0)),
      ],
      out_specs=pl.BlockSpec((br, d_out), lambda i: (i, 0)),
      out_shape=jax.ShapeDtypeStruct((n, d_out), _F32),
  )(agg1_p, deg_p, W1, b1.reshape(1, d_h), W2)

  agg2_p = agg_k(m2, src1d, dst1d)

  out = pl.pallas_call(
      _final_body,
      grid=grid,
      in_specs=[
          pl.BlockSpec((2, br, d_out), lambda i: (0, i, 0)),
          degi_spec,
          pl.BlockSpec((1, d_out), lambda i: (0, 0)),
      ],
      out_specs=pl.BlockSpec((br, d_out), lambda i: (i, 0)),
      out_shape=jax.ShapeDtypeStruct((n, d_out), _F32),
  )(agg2_p, deg_p, b2.reshape(1, d_out))

  return out


# speedup vs baseline: 1.0039x; 1.0039x over previous
import functools
import math

import jax
import jax.numpy as jnp
from jax import lax
from jax.experimental import pallas as pl
from jax.experimental.pallas import tpu as pltpu
from jax.experimental.pallas import tpu_sc as plsc

NC = 2
NS = 16
NW = NC * NS
CH = 128

_F32 = jnp.float32


def _build_deg_kernel(n_nodes, d, n_chunks):
  rows_per_sub = n_nodes // NS
  zr = 32
  assert rows_per_sub % zr == 0 and rows_per_sub % 8 == 0
  chunks_per_sub = -(-n_chunks // NS)
  mesh = plsc.VectorSubcoreMesh(core_axis_name="c", subcore_axis_name="s")

  @functools.partial(
      pl.kernel,
      out_type=jax.ShapeDtypeStruct((NC, n_nodes, d), _F32),
      mesh=mesh,
      scratch_types=(
          [pltpu.VMEM((CH,), jnp.int32) for _ in range(3)] +
          [pltpu.VMEM((CH, d), _F32),
           pltpu.VMEM((zr, d), _F32),
           pltpu.VMEM_SHARED((n_nodes, d), _F32)] +
          [pltpu.SemaphoreType.DMA for _ in range(5)]
      ),
  )
  def deg_kernel(src_hbm, dst_hbm, deg_hbm, *refs):
    idx_v = refs[0:3]
    ones_v = refs[3]
    zb = refs[4]
    deg_sh = refs[5]
    sem_i = refs[6:9]
    sem_s = refs[9:11]

    cid = lax.axis_index("c")
    sid = lax.axis_index("s")
    base = sid * rows_per_sub

    def chunk_of(t):
      return sid + t * NS

    def guard(t, fn, *a):
      @pl.when(chunk_of(t) < n_chunks)
      def _():
        fn(*a)

    def issue_idx(t, bi):
      ch = chunk_of(t)

      @pl.when(cid == 0)
      def _():
        pltpu.async_copy(src_hbm.at[pl.ds(ch * CH, CH)], idx_v[bi], sem_i[bi])

      @pl.when(cid == 1)
      def _():
        pltpu.async_copy(dst_hbm.at[pl.ds(ch * CH, CH)], idx_v[bi], sem_i[bi])

    def issue_scatter(bi, bs):
      pltpu.make_async_copy(src_hbm.at[pl.ds(0, CH)], idx_v[bi], sem_i[bi]).wait()
      pltpu.async_copy(ones_v, deg_sh.at[idx_v[bi]], sem_s[bs], add=True)

    def drain_scatter(bi, bs):
      pltpu.make_async_copy(ones_v, deg_sh.at[idx_v[bi]], sem_s[bs]).wait()

    guard(0, issue_idx, 0, 0)
    guard(1, issue_idx, 1, 1)

    def fill_ones(i, c):
      for j in range(d // 16):
        ones_v[i, pl.ds(j * 16, 16)] = jnp.ones((16,), _F32)
      return c
    lax.fori_loop(0, CH, fill_ones, 0)

    def fill_zb(i, c):
      for j in range(d // 16):
        zb[i, pl.ds(j * 16, 16)] = jnp.zeros((16,), _F32)
      return c
    lax.fori_loop(0, zr, fill_zb, 0)

    def zero_slice(t, c):
      pltpu.sync_copy(zb, deg_sh.at[pl.ds(base + t * zr, zr)])
      return c
    lax.fori_loop(0, rows_per_sub // zr, zero_slice, 0)
    plsc.subcore_barrier()

    def step(t, bi, bs, drain=True):
      guard(t, issue_scatter, bi, bs)
      if drain:
        guard(t - 1, drain_scatter, (bi - 1) % 3, (bs - 1) % 2)
      guard(t + 2, issue_idx, t + 2, (bi + 2) % 3)

    step(0, 0, 0, drain=False)
    step(1, 1, 1)

    def edge_loop(i, c):
      t0 = 2 + i * 6
      for k in range(6):
        t = t0 + k

        @pl.when(chunk_of(t - 1) < n_chunks)
        def _(t=t, k=k):
          step(t, (2 + k) % 3, (2 + k) % 2)
      return c
    nsteps = chunks_per_sub - 1
    lax.fori_loop(0, (nsteps + 5) // 6, edge_loop, 0)
    plsc.subcore_barrier()

    pltpu.sync_copy(deg_sh.at[pl.ds(base, rows_per_sub)],
                    deg_hbm.at[cid, pl.ds(base, rows_per_sub)])

  return deg_kernel


def _build_agg_kernel(n_nodes, d, n_chunks):
  rows_per_sub = n_nodes // NS
  zr = 32
  assert rows_per_sub % zr == 0 and rows_per_sub % 8 == 0
  chunks_per_worker = -(-n_chunks // NW)
  mesh = plsc.VectorSubcoreMesh(core_axis_name="c", subcore_axis_name="s")

  @functools.partial(
      pl.kernel,
      out_type=jax.ShapeDtypeStruct((NC, n_nodes, d), _F32),
      mesh=mesh,
      scratch_types=[
          pltpu.VMEM((CH,), jnp.int32),
          pltpu.VMEM((CH,), jnp.int32),
          pltpu.VMEM((CH,), jnp.int32),
          pltpu.VMEM((CH,), jnp.int32),
          pltpu.VMEM((CH, d), _F32),
          pltpu.VMEM((CH, d), _F32),
          pltpu.VMEM((zr, d), _F32),
          pltpu.VMEM_SHARED((n_nodes, d), _F32),
          pltpu.SemaphoreType.DMA,
          pltpu.SemaphoreType.DMA,
          pltpu.SemaphoreType.DMA,
          pltpu.SemaphoreType.DMA,
      ],
  )
  def agg_kernel(x_hbm, src_hbm, dst_hbm, agg_hbm,
                 idx_s, idx_s2, idx_d, idx_d2, rows_v, rows_v2, zb, acc_sh,
                 sem_a, sem_b, sem_ia, sem_ib):
    cid = lax.axis_index("c")
    sid = lax.axis_index("s")
    wid = sid * NC + cid
    base = sid * rows_per_sub

    def issue_idx(ch, idxb, dstb, sem):
      pltpu.async_copy(src_hbm.at[pl.ds(ch * CH, CH)], idxb, sem)
      pltpu.async_copy(dst_hbm.at[pl.ds(ch * CH, CH)], dstb, sem)

    def wait_idx(idxb, dstb, sem):
      pltpu.make_async_copy(src_hbm.at[pl.ds(0, CH)], idxb, sem).wait()
      pltpu.make_async_copy(src_hbm.at[pl.ds(0, CH)], dstb, sem).wait()

    @pl.when(wid < n_chunks)
    def _():
      issue_idx(wid, idx_s, idx_d, sem_ia)

    @pl.when(wid + NW < n_chunks)
    def _():
      issue_idx(wid + NW, idx_s2, idx_d2, sem_ib)

    def fill_zb(i, c):
      for j in range(d // 16):
        zb[i, pl.ds(j * 16, 16)] = jnp.zeros((16,), _F32)
      return c
    lax.fori_loop(0, zr, fill_zb, 0)

    def zero_slice(t, c):
      pltpu.sync_copy(zb, acc_sh.at[pl.ds(base + t * zr, zr)])
      return c
    lax.fori_loop(0, rows_per_sub // zr, zero_slice, 0)

    @pl.when(wid < n_chunks)
    def _():
      wait_idx(idx_s, idx_d, sem_ia)
      pltpu.async_copy(x_hbm.at[idx_s], rows_v, sem_a)
    plsc.subcore_barrier()

    def step(c0, c1, c2, idxA, dstA, rowsA, semA, semIA,
             idxB, dstB, rowsB, semB, semIB):
      @pl.when(c1 < n_chunks)
      def _():
        wait_idx(idxB, dstB, semIB)
        pltpu.async_copy(x_hbm.at[idxB], rowsB, semB)

      @pl.when(c0 < n_chunks)
      def _():
        pltpu.make_async_copy(x_hbm.at[idxA], rowsA, semA).wait()
        pltpu.sync_copy(rowsA, acc_sh.at[dstA], add=True)

      @pl.when(c2 < n_chunks)
      def _():
        issue_idx(c2, idxA, dstA, semIA)

    def edge_loop(j, c):
      c0 = wid + (2 * j) * NW
      step(c0, c0 + NW, c0 + 2 * NW,
           idx_s, idx_d, rows_v, sem_a, sem_ia,
           idx_s2, idx_d2, rows_v2, sem_b, sem_ib)
      step(c0 + NW, c0 + 2 * NW, c0 + 3 * NW,
           idx_s2, idx_d2, rows_v2, sem_b, sem_ib,
           idx_s, idx_d, rows_v, sem_a, sem_ia)
      return c
    lax.fori_loop(0, (chunks_per_worker + 1) // 2, edge_loop, 0)
    plsc.subcore_barrier()

    pltpu.sync_copy(acc_sh.at[pl.ds(base, rows_per_sub)],
                    agg_hbm.at[cid, pl.ds(base, rows_per_sub)])

  return agg_kernel


def _scale_body(feat_ref, dego_ref, out_ref):
  deg = dego_ref[0, :, 0:1] + 1.0
  out_ref[...] = feat_ref[...] * lax.rsqrt(deg)


def _gelu_exact(x):
  return 0.5 * x * (1.0 + lax.erf(x * (1.0 / math.sqrt(2.0))))


def _mid_body(agg1_ref, xs_ref, deg_ref, w1_ref, b1_ref, w2_ref,
              out_ref):
  a1 = agg1_ref[0] + agg1_ref[1] + xs_ref[...]
  ii = lax.rsqrt(deg_ref[1, :, 0:1] + 1.0)
  oo = lax.rsqrt(deg_ref[0, :, 0:1] + 1.0)
  h = jnp.dot(a1, w1_ref[...], preferred_element_type=_F32) * ii + b1_ref[...]
  h = _gelu_exact(h)
  out_ref[...] = jnp.dot(h * oo, w2_ref[...], preferred_element_type=_F32)


def _final_body(agg2_ref, m2_ref, degi_ref, b2_ref, out_ref):
  ii = lax.rsqrt(degi_ref[0, :, 0:1] + 1.0)
  y = (agg2_ref[0] + agg2_ref[1] + m2_ref[...]) * ii + b2_ref[...]
  nrm = jnp.sqrt(jnp.sum(y * y, axis=1, keepdims=True))
  out_ref[...] = y / jnp.maximum(nrm, 1e-12)


def kernel(features, edge_index, W1, b1, W2, b2):
  n, d_in = features.shape
  d_h = W1.shape[1]
  d_out = W2.shape[1]
  e = edge_index.shape[1]
  assert e % CH == 0 and n % NS == 0
  n_chunks = e // CH

  src1d = edge_index[0]
  dst1d = edge_index[1]
  np_pad = -(-n // (NS * 64)) * (NS * 64)

  deg_k = _build_deg_kernel(np_pad, d_in, n_chunks)
  agg_k = _build_agg_kernel(np_pad, d_in, n_chunks)

  deg_p = deg_k(src1d, dst1d)

  br = 400
  grid = (n // br,)
  dego_spec = pl.BlockSpec((1, br, d_in), lambda i: (0, i, 0))
  degi_spec = pl.BlockSpec((1, br, d_in), lambda i: (1, i, 0))
  degb_spec = pl.BlockSpec((2, br, d_in), lambda i: (0, i, 0))
  row_spec = pl.BlockSpec((br, d_in), lambda i: (i, 0))
  agg_spec = pl.BlockSpec((2, br, d_in), lambda i: (0, i, 0))

  xs = pl.pallas_call(
      _scale_body,
      grid=grid,
      in_specs=[row_spec, dego_spec],
      out_specs=row_spec,
      out_shape=jax.ShapeDtypeStruct((n, d_in), _F32),
  )(features, deg_p)

  agg1_p = agg_k(xs, src1d, dst1d)

  m2 = pl.pallas_call(
      _mid_body,
      grid=grid,
      in_specs=[
          agg_spec,
          row_spec,
          degb_spec,
          pl.BlockSpec((d_in, d_h), lambda i: (0, 0)),
          pl.BlockSpec((1, d_h), lambda i: (0, 0)),
          pl.BlockSpec((d_h, d_out), lambda i: (0, 0)),
      ],
      out_specs=pl.BlockSpec((br, d_out), lambda i: (i, 0)),
      out_shape=jax.ShapeDtypeStruct((n, d_out), _F32),
  )(agg1_p, xs, deg_p, W1, b1.reshape(1, d_h), W2)

  agg2_p = agg_k(m2, src1d, dst1d)

  out = pl.pallas_call(
      _final_body,
      grid=grid,
      in_specs=[
          pl.BlockSpec((2, br, d_out), lambda i: (0, i, 0)),
          pl.BlockSpec((br, d_out), lambda i: (i, 0)),
          degi_spec,
          pl.BlockSpec((1, d_out), lambda i: (0, 0)),
      ],
      out_specs=pl.BlockSpec((br, d_out), lambda i: (i, 0)),
      out_shape=jax.ShapeDtypeStruct((n, d_out), _F32),
  )(agg2_p, m2, deg_p, b2.reshape(1, d_out))

  return out
